# baseline (speedup 1.0000x reference)
import functools
import jax
import jax.numpy as jnp
from jax import lax
from jax.experimental import pallas as pl
from jax.experimental.pallas import tpu as pltpu
from jax.experimental.pallas import tpu_sc as plsc

N = 10000
E = 160000
H = 256
NC = 2
NS = 16
NW = NC * NS
GCH = 128
NSLAB = 5
E_S = E // NSLAB
NCH128 = E // 128
N_PAD = 10112
ROWS_PT = N_PAD // NS
bf16 = jnp.bfloat16
f32 = jnp.float32

_lazy_cache = {}


def _sc_mesh():
    if "mesh" not in _lazy_cache:
        _lazy_cache["mesh"] = plsc.VectorSubcoreMesh(
            core_axis_name="c", subcore_axis_name="s",
            num_cores=NC, num_subcores=NS)
    return _lazy_cache["mesh"]


def _sc_kernel(name, **kw):
    def deco(body):
        def call(*args):
            if name not in _lazy_cache:
                _lazy_cache[name] = pl.kernel(body, mesh=_sc_mesh(), **kw)
            return _lazy_cache[name](*args)
        return call
    return deco


def _pack_cols(acc):
    lo = acc[:, :128].astype(bf16).astype(f32)
    hi = acc[:, 128:].astype(bf16).astype(f32)
    lo_b = lax.bitcast_convert_type(lo, jnp.uint32)
    hi_b = lax.bitcast_convert_type(hi, jnp.uint32)
    return lax.bitcast_convert_type(hi_b | (lo_b >> 16), jnp.int32)


def _unpack_cols(w):
    u = lax.bitcast_convert_type(w, jnp.uint32)
    lo = lax.bitcast_convert_type(u << 16, f32)
    hi = lax.bitcast_convert_type(u & jnp.uint32(0xFFFF0000), f32)
    return lo, hi


def _enc_ab_body(x_ref, we_ref, be_ref, w1t_ref, w1b_ref, b1_ref, a_ref, b_ref):
    h = jnp.maximum(
        jnp.dot(x_ref[...].astype(bf16), we_ref[...],
                preferred_element_type=f32) + be_ref[...], 0.0)
    hb = h.astype(bf16)
    a_ref[...] = _pack_cols(jnp.dot(hb, w1t_ref[...], preferred_element_type=f32)
                            + b1_ref[...])
    b_ref[...] = _pack_cols(jnp.dot(hb, w1b_ref[...], preferred_element_type=f32))


def _enc_ab(x, we, be, w1t, w1b, b1):
    blk = 400
    grid = (N // blk,)
    return pl.pallas_call(
        _enc_ab_body,
        grid=grid,
        in_specs=[
            pl.BlockSpec((blk, H), lambda i: (i, 0)),
            pl.BlockSpec((H, H), lambda i: (0, 0)),
            pl.BlockSpec((1, H), lambda i: (0, 0)),
            pl.BlockSpec((H, H), lambda i: (0, 0)),
            pl.BlockSpec((H, H), lambda i: (0, 0)),
            pl.BlockSpec((1, H), lambda i: (0, 0)),
        ],
        out_specs=[
            pl.BlockSpec((blk, 128), lambda i: (i, 0)),
            pl.BlockSpec((blk, 128), lambda i: (i, 0)),
        ],
        out_shape=[
            jax.ShapeDtypeStruct((N, 128), jnp.int32),
            jax.ShapeDtypeStruct((N, 128), jnp.int32),
        ],
    )(x, we, be, w1t, w1b, b1)


def _mlp_body(ga_ref, gb_ref, w2_ref, b2_ref, m2a_ref, m2b_ref):
    la, ha = _unpack_cols(ga_ref[...])
    lb, hb = _unpack_cols(gb_ref[...])
    m = jnp.concatenate([jnp.maximum(la + lb, 0.0),
                         jnp.maximum(ha + hb, 0.0)], axis=1).astype(bf16)
    acc = jnp.dot(m, w2_ref[...], preferred_element_type=f32) + b2_ref[...]
    m2 = jnp.maximum(acc, 0.0)
    m2a_ref[...] = m2[:, :128]
    m2b_ref[...] = m2[:, 128:]


def _mlp(ga, gb, w2, b2):
    blk = 640
    grid = (ga.shape[0] // blk,)
    return pl.pallas_call(
        _mlp_body,
        grid=grid,
        in_specs=[
            pl.BlockSpec((blk, 128), lambda i: (i, 0)),
            pl.BlockSpec((blk, 128), lambda i: (i, 0)),
            pl.BlockSpec((H, H), lambda i: (0, 0)),
            pl.BlockSpec((1, H), lambda i: (0, 0)),
        ],
        out_specs=[
            pl.BlockSpec((blk, 128), lambda i: (i, 0)),
            pl.BlockSpec((blk, 128), lambda i: (i, 0)),
        ],
        out_shape=[
            jax.ShapeDtypeStruct((ga.shape[0], 128), f32),
            jax.ShapeDtypeStruct((ga.shape[0], 128), f32),
        ],
    )(ga, gb, w2, b2)


def _agg_h(s0_ref, s1_ref, c0_ref, c1_ref):
    csum = jnp.sum(c0_ref[...] + c1_ref[...], axis=1, keepdims=True)
    r = 128.0 / jnp.maximum(csum, 128.0)
    return s0_ref[...] * r, s1_ref[...] * r


def _agg_ab_body(s0_ref, s1_ref, c0_ref, c1_ref,
                 w1t_hi, w1t_lo, w1b_hi, w1b_lo, b1_ref, a_ref, b_ref):
    h0, h1 = _agg_h(s0_ref, s1_ref, c0_ref, c1_ref)
    h0 = h0.astype(bf16)
    h1 = h1.astype(bf16)
    a_ref[...] = _pack_cols(jnp.dot(h0, w1t_hi[...], preferred_element_type=f32)
                            + jnp.dot(h1, w1t_lo[...], preferred_element_type=f32)
                            + b1_ref[...])
    b_ref[...] = _pack_cols(jnp.dot(h0, w1b_hi[...], preferred_element_type=f32)
                            + jnp.dot(h1, w1b_lo[...], preferred_element_type=f32))


def _agg_ab(s0, s1, c0, c1, w1t_hi, w1t_lo, w1b_hi, w1b_lo, b1):
    blk = 400
    grid = (N // blk,)
    return pl.pallas_call(
        _agg_ab_body,
        grid=grid,
        in_specs=[
            pl.BlockSpec((blk, 128), lambda i: (i, 0)),
            pl.BlockSpec((blk, 128), lambda i: (i, 0)),
            pl.BlockSpec((blk, 128), lambda i: (i, 0)),
            pl.BlockSpec((blk, 128), lambda i: (i, 0)),
            pl.BlockSpec((128, H), lambda i: (0, 0)),
            pl.BlockSpec((128, H), lambda i: (0, 0)),
            pl.BlockSpec((128, H), lambda i: (0, 0)),
            pl.BlockSpec((128, H), lambda i: (0, 0)),
            pl.BlockSpec((1, H), lambda i: (0, 0)),
        ],
        out_specs=[
            pl.BlockSpec((blk, 128), lambda i: (i, 0)),
            pl.BlockSpec((blk, 128), lambda i: (i, 0)),
        ],
        out_shape=[
            jax.ShapeDtypeStruct((N, 128), jnp.int32),
            jax.ShapeDtypeStruct((N, 128), jnp.int32),
        ],
    )(s0, s1, c0, c1, w1t_hi, w1t_lo, w1b_hi, w1b_lo, b1)


def _dec_body(s0_ref, s1_ref, c0_ref, c1_ref, wd_hi, wd_lo, bd_ref, o_ref):
    h0, h1 = _agg_h(s0_ref, s1_ref, c0_ref, c1_ref)
    o_ref[...] = (jnp.dot(h0, wd_hi[...], preferred_element_type=f32)
                  + jnp.dot(h1, wd_lo[...], preferred_element_type=f32)
                  + bd_ref[...])


def _dec(s0, s1, c0, c1, wd_hi, wd_lo, bd):
    blk = 400
    grid = (N // blk,)
    return pl.pallas_call(
        _dec_body,
        grid=grid,
        in_specs=[
            pl.BlockSpec((blk, 128), lambda i: (i, 0)),
            pl.BlockSpec((blk, 128), lambda i: (i, 0)),
            pl.BlockSpec((blk, 128), lambda i: (i, 0)),
            pl.BlockSpec((blk, 128), lambda i: (i, 0)),
            pl.BlockSpec((128, H), lambda i: (0, 0)),
            pl.BlockSpec((128, H), lambda i: (0, 0)),
            pl.BlockSpec((1, H), lambda i: (0, 0)),
        ],
        out_specs=pl.BlockSpec((blk, H), lambda i: (i, 0)),
        out_shape=jax.ShapeDtypeStruct((N, H), f32),
    )(s0, s1, c0, c1, wd_hi, wd_lo, bd)


@_sc_kernel(
    "cnt",
    out_type=jax.ShapeDtypeStruct((NC, N_PAD, 128), f32),
    scratch_types=[
        pltpu.VMEM((128,), jnp.int32),
        pltpu.VMEM((128, 128), f32),
        pltpu.VMEM_SHARED((N_PAD, 128), f32),
    ],
)
def _cnt_kernel(dst_hbm, out_hbm, idx_v, rows_v, acc_sh):
    cid = lax.axis_index("c")
    sid = lax.axis_index("s")

    def zb(i, _):
        for k in range(8):
            rows_v[i, pl.ds(k * 16, 16)] = jnp.zeros((16,), f32)
        return 0
    lax.fori_loop(0, 128, zb, 0)
    base = sid * ROWS_PT
    for off, ln in ((0, 128), (128, 128), (256, 128), (384, 128), (512, 120)):
        pltpu.sync_copy(rows_v.at[pl.ds(0, ln)],
                        acc_sh.at[pl.ds(base + off, ln)])

    plsc.subcore_barrier()

    def ob(i, _):
        for k in range(8):
            rows_v[i, pl.ds(k * 16, 16)] = jnp.ones((16,), f32)
        return 0
    lax.fori_loop(0, 128, ob, 0)

    wid = sid * NC + cid
    n_j = (NCH128 // NW) + jnp.where(wid < NCH128 - (NCH128 // NW) * NW, 1, 0)

    def body(j, _):
        e_base = (wid + j * NW) * 128
        pltpu.sync_copy(dst_hbm.at[pl.ds(e_base, 128)], idx_v)
        pltpu.sync_copy(rows_v, acc_sh.at[idx_v], add=True)
        return 0
    lax.fori_loop(0, n_j, body, 0)

    plsc.subcore_barrier()

    pltpu.sync_copy(acc_sh.at[pl.ds(base, ROWS_PT)],
                    out_hbm.at[cid].at[pl.ds(base, ROWS_PT)])


@_sc_kernel(
    "gather",
    out_type=[
        jax.ShapeDtypeStruct((E_S, 128), jnp.int32),
        jax.ShapeDtypeStruct((E_S, 128), jnp.int32),
    ],
    scratch_types=[
        pltpu.VMEM((128,), jnp.int32),
        pltpu.VMEM((128,), jnp.int32),
        pltpu.VMEM((128,), jnp.int32),
        pltpu.VMEM((128,), jnp.int32),
        pltpu.VMEM((GCH, 128), jnp.int32),
        pltpu.VMEM((GCH, 128), jnp.int32),
        pltpu.VMEM((GCH, 128), jnp.int32),
        pltpu.VMEM((GCH, 128), jnp.int32),
        pltpu.SemaphoreType.DMA,
        pltpu.SemaphoreType.DMA,
        pltpu.SemaphoreType.DMA,
        pltpu.SemaphoreType.DMA,
    ],
)
def _gather_kernel(a_hbm, b_hbm, dst_hbm, src_hbm, ga_hbm, gb_hbm,
                   ixd0, ixd1, ixs0, ixs1, ad0, ad1, bs0, bs1,
                   sg0, sg1, sw0, sw1):
    cid = lax.axis_index("c")
    sid = lax.axis_index("s")
    wid = sid * NC + cid
    ngc = dst_hbm.shape[0] // GCH
    n_t = (ngc // NW) + jnp.where(wid < ngc - (ngc // NW) * NW, 1, 0)
    kmax = -(-ngc // NW)

    ixd = (ixd0, ixd1)
    ixs = (ixs0, ixs1)
    ad = (ad0, ad1)
    bs = (bs0, bs1)
    sg = (sg0, sg1)
    sw = (sw0, sw1)

    def drain(sem, n):
        for _ in range(n):
            pltpu.make_async_copy(ga_hbm.at[pl.ds(0, GCH)], ad0, sem).wait()

    def slot(k, p, fire_new):
        @pl.when((k >= 2) & (k - 2 < n_t))
        def _():
            drain(sw[p], 2)

        if fire_new:
            @pl.when(k < n_t)
            def _():
                base_e = (wid + k * NW) * GCH
                pltpu.sync_copy(dst_hbm.at[pl.ds(base_e, GCH)], ixd[p])
                pltpu.sync_copy(src_hbm.at[pl.ds(base_e, GCH)], ixs[p])
                pltpu.async_copy(a_hbm.at[ixd[p]], ad[p], sg[p])
                pltpu.async_copy(b_hbm.at[ixs[p]], bs[p], sg[p])

        q = 1 - p

        @pl.when((k >= 1) & (k - 1 < n_t))
        def _():
            drain(sg[q], 2)
            base_e = (wid + (k - 1) * NW) * GCH
            pltpu.async_copy(ad[q], ga_hbm.at[pl.ds(base_e, GCH)], sw[q])
            pltpu.async_copy(bs[q], gb_hbm.at[pl.ds(base_e, GCH)], sw[q])

    def body(u, _):
        slot(2 * u, 0, True)
        slot(2 * u + 1, 1, True)
        return 0

    npair = (kmax + 1) // 2
    lax.fori_loop(0, npair, body, 0)
    slot(2 * npair, 0, False)
    slot(2 * npair + 1, 1, False)


@_sc_kernel(
    "scatter",
    out_type=[
        jax.ShapeDtypeStruct((N, 128), f32),
        jax.ShapeDtypeStruct((N, 128), f32),
    ],
    scratch_types=[
        pltpu.VMEM((128,), jnp.int32),
        pltpu.VMEM((128,), jnp.int32),
        pltpu.VMEM((128, 128), f32),
        pltpu.VMEM((128, 128), f32),
        pltpu.VMEM_SHARED((N_PAD, 128), f32),
        pltpu.SemaphoreType.DMA,
        pltpu.SemaphoreType.DMA,
        pltpu.SemaphoreType.DMA,
        pltpu.SemaphoreType.DMA,
    ],
)
def _scatter_kernel(m2a_hbm, m2b_hbm, dst_hbm, s0_hbm, s1_hbm,
                    ix0, ix1, rv0, rv1, acc_sh, sl0, sl1, ss0, ss1):
    cid = lax.axis_index("c")
    sid = lax.axis_index("s")

    ix = (ix0, ix1)
    rv = (rv0, rv1)
    sl = (sl0, sl1)
    ss = (ss0, ss1)

    def zb(i, _):
        for k in range(8):
            rv0[i, pl.ds(k * 16, 16)] = jnp.zeros((16,), f32)
        return 0
    lax.fori_loop(0, 128, zb, 0)
    base = sid * ROWS_PT
    for off, ln in ((0, 128), (128, 128), (256, 128), (384, 128), (512, 120)):
        pltpu.sync_copy(rv0.at[pl.ds(0, ln)],
                        acc_sh.at[pl.ds(base + off, ln)])

    plsc.subcore_barrier()

    nch = m2a_hbm.shape[0] // 128
    n_j = (nch // NS) + jnp.where(sid < nch - (nch // NS) * NS, 1, 0)
    kmax = -(-nch // NS)

    def work(m2_ref):
        def drain_rows(sem):
            pltpu.make_async_copy(m2_ref.at[pl.ds(0, 128)], rv0, sem).wait()

        def drain_idx(sem):
            pltpu.make_async_copy(dst_hbm.at[pl.ds(0, 128)], ix0, sem).wait()

        def slot(k, p, fire_new):
            @pl.when((k >= 2) & (k - 2 < n_j))
            def _():
                drain_rows(ss[p])

            if fire_new:
                @pl.when(k < n_j)
                def _():
                    e_base = (sid + k * NS) * 128
                    pltpu.async_copy(dst_hbm.at[pl.ds(e_base, 128)], ix[p], sl[p])
                    pltpu.async_copy(m2_ref.at[pl.ds(e_base, 128)], rv[p], sl[p])

            q = 1 - p

            @pl.when((k >= 1) & (k - 1 < n_j))
            def _():
                drain_idx(sl[q])
                drain_rows(sl[q])
                pltpu.async_copy(rv[q], acc_sh.at[ix[q]], ss[q], add=True)

        def body(u, _):
            slot(2 * u, 0, True)
            slot(2 * u + 1, 1, True)
            return 0

        npair = (kmax + 1) // 2
        lax.fori_loop(0, npair, body, 0)
        slot(2 * npair, 0, False)
        slot(2 * npair + 1, 1, False)

    @pl.when(cid == 0)
    def _():
        work(m2a_hbm)

    @pl.when(cid == 1)
    def _():
        work(m2b_hbm)

    plsc.subcore_barrier()

    def writeback(s_hbm):
        @pl.when(sid < NS - 1)
        def _():
            pltpu.sync_copy(acc_sh.at[pl.ds(base, ROWS_PT)],
                            s_hbm.at[pl.ds(base, ROWS_PT)])

        @pl.when(sid == NS - 1)
        def _():
            ln = N - (NS - 1) * ROWS_PT
            pltpu.sync_copy(acc_sh.at[pl.ds((NS - 1) * ROWS_PT, ln)],
                            s_hbm.at[pl.ds((NS - 1) * ROWS_PT, ln)])

    @pl.when(cid == 0)
    def _():
        writeback(s0_hbm)

    @pl.when(cid == 1)
    def _():
        writeback(s1_hbm)


def kernel(x, edge_index, W_enc, b_enc,
           l0_W1, l0_b1, l0_W2, l0_b2,
           l1_W1, l1_b1, l1_W2, l1_b2,
           l2_W1, l2_b1, l2_W2, l2_b2,
           W_dec, b_dec):
    src1 = edge_index[0]
    dst1 = edge_index[1]

    layers = [(l0_W1, l0_b1, l0_W2, l0_b2),
              (l1_W1, l1_b1, l1_W2, l1_b2),
              (l2_W1, l2_b1, l2_W2, l2_b2)]

    craw = _cnt_kernel(dst1)
    c0 = craw[0, :N]
    c1 = craw[1, :N]

    A, B = _enc_ab(x, W_enc.astype(bf16), b_enc.reshape(1, H),
                   l0_W1[:H].astype(bf16), l0_W1[H:].astype(bf16),
                   l0_b1.reshape(1, H))

    dsts = [lax.slice_in_dim(dst1, s * E_S, (s + 1) * E_S) for s in range(NSLAB)]
    srcs = [lax.slice_in_dim(src1, s * E_S, (s + 1) * E_S) for s in range(NSLAB)]

    out = None
    for l, (W1, b1, W2, b2) in enumerate(layers):
        w2b = W2.astype(bf16)
        b2r = b2.reshape(1, H)
        parts = []
        for s in range(NSLAB):
            ga_i, gb_i = _gather_kernel(A, B, dsts[s], srcs[s])
            m2a, m2b = _mlp(ga_i, gb_i, w2b, b2r)
            parts.append(_scatter_kernel(m2a, m2b, dsts[s]))
        s0 = parts[0][0]
        s1 = parts[0][1]
        for q in range(1, NSLAB):
            s0 = s0 + parts[q][0]
            s1 = s1 + parts[q][1]
        if l == 2:
            out = _dec(s0, s1, c0, c1, W_dec[:128], W_dec[128:],
                       b_dec.reshape(1, H))
        else:
            nW1, nb1 = layers[l + 1][0], layers[l + 1][1]
            A, B = _agg_ab(s0, s1, c0, c1,
                           nW1[:128].astype(bf16), nW1[128:H].astype(bf16),
                           nW1[H:H + 128].astype(bf16), nW1[H + 128:].astype(bf16),
                           nb1.reshape(1, H))
    return out


# speedup vs baseline: 1.0172x; 1.0172x over previous
import functools
import jax
import jax.numpy as jnp
from jax import lax
from jax.experimental import pallas as pl
from jax.experimental.pallas import tpu as pltpu
from jax.experimental.pallas import tpu_sc as plsc

N = 10000
E = 160000
H = 256
NC = 2
NS = 16
NW = NC * NS
GCH = 128
NSLAB = 2
E_S = E // NSLAB
NCH128 = E // 128
N_PAD = 10112
ROWS_PT = N_PAD // NS
bf16 = jnp.bfloat16
f32 = jnp.float32

_lazy_cache = {}


def _sc_mesh():
    if "mesh" not in _lazy_cache:
        _lazy_cache["mesh"] = plsc.VectorSubcoreMesh(
            core_axis_name="c", subcore_axis_name="s",
            num_cores=NC, num_subcores=NS)
    return _lazy_cache["mesh"]


def _sc_kernel(name, **kw):
    def deco(body):
        def call(*args):
            if name not in _lazy_cache:
                _lazy_cache[name] = pl.kernel(body, mesh=_sc_mesh(), **kw)
            return _lazy_cache[name](*args)
        return call
    return deco


def _pack_cols(acc):
    lo = acc[:, :128].astype(bf16).astype(f32)
    hi = acc[:, 128:].astype(bf16).astype(f32)
    lo_b = lax.bitcast_convert_type(lo, jnp.uint32)
    hi_b = lax.bitcast_convert_type(hi, jnp.uint32)
    return lax.bitcast_convert_type(hi_b | (lo_b >> 16), jnp.int32)


def _unpack_cols(w):
    u = lax.bitcast_convert_type(w, jnp.uint32)
    lo = lax.bitcast_convert_type(u << 16, f32)
    hi = lax.bitcast_convert_type(u & jnp.uint32(0xFFFF0000), f32)
    return lo, hi


def _enc_ab_body(x_ref, we_ref, be_ref, w1t_ref, w1b_ref, b1_ref, a_ref, b_ref):
    h = jnp.maximum(
        jnp.dot(x_ref[...].astype(bf16), we_ref[...],
                preferred_element_type=f32) + be_ref[...], 0.0)
    hb = h.astype(bf16)
    a_ref[...] = _pack_cols(jnp.dot(hb, w1t_ref[...], preferred_element_type=f32)
                            + b1_ref[...])
    b_ref[...] = _pack_cols(jnp.dot(hb, w1b_ref[...], preferred_element_type=f32))


def _enc_ab(x, we, be, w1t, w1b, b1):
    blk = 400
    grid = (N // blk,)
    return pl.pallas_call(
        _enc_ab_body,
        grid=grid,
        in_specs=[
            pl.BlockSpec((blk, H), lambda i: (i, 0)),
            pl.BlockSpec((H, H), lambda i: (0, 0)),
            pl.BlockSpec((1, H), lambda i: (0, 0)),
            pl.BlockSpec((H, H), lambda i: (0, 0)),
            pl.BlockSpec((H, H), lambda i: (0, 0)),
            pl.BlockSpec((1, H), lambda i: (0, 0)),
        ],
        out_specs=[
            pl.BlockSpec((blk, 128), lambda i: (i, 0)),
            pl.BlockSpec((blk, 128), lambda i: (i, 0)),
        ],
        out_shape=[
            jax.ShapeDtypeStruct((N, 128), jnp.int32),
            jax.ShapeDtypeStruct((N, 128), jnp.int32),
        ],
    )(x, we, be, w1t, w1b, b1)


def _mlp_body(ga_ref, gb_ref, w2_ref, b2_ref, m2a_ref, m2b_ref):
    la, ha = _unpack_cols(ga_ref[...])
    lb, hb = _unpack_cols(gb_ref[...])
    m = jnp.concatenate([jnp.maximum(la + lb, 0.0),
                         jnp.maximum(ha + hb, 0.0)], axis=1).astype(bf16)
    acc = jnp.dot(m, w2_ref[...], preferred_element_type=f32) + b2_ref[...]
    m2 = jnp.maximum(acc, 0.0)
    m2a_ref[...] = m2[:, :128]
    m2b_ref[...] = m2[:, 128:]


def _mlp(ga, gb, w2, b2):
    blk = 640
    grid = (ga.shape[0] // blk,)
    return pl.pallas_call(
        _mlp_body,
        grid=grid,
        in_specs=[
            pl.BlockSpec((blk, 128), lambda i: (i, 0)),
            pl.BlockSpec((blk, 128), lambda i: (i, 0)),
            pl.BlockSpec((H, H), lambda i: (0, 0)),
            pl.BlockSpec((1, H), lambda i: (0, 0)),
        ],
        out_specs=[
            pl.BlockSpec((blk, 128), lambda i: (i, 0)),
            pl.BlockSpec((blk, 128), lambda i: (i, 0)),
        ],
        out_shape=[
            jax.ShapeDtypeStruct((ga.shape[0], 128), f32),
            jax.ShapeDtypeStruct((ga.shape[0], 128), f32),
        ],
    )(ga, gb, w2, b2)


def _agg_h(s0a_ref, s0b_ref, s1a_ref, s1b_ref, c0_ref, c1_ref):
    csum = jnp.sum(c0_ref[...] + c1_ref[...], axis=1, keepdims=True)
    r = 128.0 / jnp.maximum(csum, 128.0)
    return ((s0a_ref[...] + s0b_ref[...]) * r,
            (s1a_ref[...] + s1b_ref[...]) * r)


def _agg_ab_body(s0a, s0b, s1a, s1b, c0_ref, c1_ref,
                 w1t_hi, w1t_lo, w1b_hi, w1b_lo, b1_ref, a_ref, b_ref):
    h0, h1 = _agg_h(s0a, s0b, s1a, s1b, c0_ref, c1_ref)
    h0 = h0.astype(bf16)
    h1 = h1.astype(bf16)
    a_ref[...] = _pack_cols(jnp.dot(h0, w1t_hi[...], preferred_element_type=f32)
                            + jnp.dot(h1, w1t_lo[...], preferred_element_type=f32)
                            + b1_ref[...])
    b_ref[...] = _pack_cols(jnp.dot(h0, w1b_hi[...], preferred_element_type=f32)
                            + jnp.dot(h1, w1b_lo[...], preferred_element_type=f32))


def _agg_ab(s0a, s0b, s1a, s1b, c0, c1, w1t_hi, w1t_lo, w1b_hi, w1b_lo, b1):
    blk = 400
    grid = (N // blk,)
    return pl.pallas_call(
        _agg_ab_body,
        grid=grid,
        in_specs=[
            pl.BlockSpec((blk, 128), lambda i: (i, 0)),
            pl.BlockSpec((blk, 128), lambda i: (i, 0)),
            pl.BlockSpec((blk, 128), lambda i: (i, 0)),
            pl.BlockSpec((blk, 128), lambda i: (i, 0)),
            pl.BlockSpec((blk, 128), lambda i: (i, 0)),
            pl.BlockSpec((blk, 128), lambda i: (i, 0)),
            pl.BlockSpec((128, H), lambda i: (0, 0)),
            pl.BlockSpec((128, H), lambda i: (0, 0)),
            pl.BlockSpec((128, H), lambda i: (0, 0)),
            pl.BlockSpec((128, H), lambda i: (0, 0)),
            pl.BlockSpec((1, H), lambda i: (0, 0)),
        ],
        out_specs=[
            pl.BlockSpec((blk, 128), lambda i: (i, 0)),
            pl.BlockSpec((blk, 128), lambda i: (i, 0)),
        ],
        out_shape=[
            jax.ShapeDtypeStruct((N, 128), jnp.int32),
            jax.ShapeDtypeStruct((N, 128), jnp.int32),
        ],
    )(s0a, s0b, s1a, s1b, c0, c1, w1t_hi, w1t_lo, w1b_hi, w1b_lo, b1)


def _dec_body(s0a, s0b, s1a, s1b, c0_ref, c1_ref, wd_hi, wd_lo, bd_ref, o_ref):
    h0, h1 = _agg_h(s0a, s0b, s1a, s1b, c0_ref, c1_ref)
    o_ref[...] = (jnp.dot(h0, wd_hi[...], preferred_element_type=f32)
                  + jnp.dot(h1, wd_lo[...], preferred_element_type=f32)
                  + bd_ref[...])


def _dec(s0a, s0b, s1a, s1b, c0, c1, wd_hi, wd_lo, bd):
    blk = 400
    grid = (N // blk,)
    return pl.pallas_call(
        _dec_body,
        grid=grid,
        in_specs=[
            pl.BlockSpec((blk, 128), lambda i: (i, 0)),
            pl.BlockSpec((blk, 128), lambda i: (i, 0)),
            pl.BlockSpec((blk, 128), lambda i: (i, 0)),
            pl.BlockSpec((blk, 128), lambda i: (i, 0)),
            pl.BlockSpec((blk, 128), lambda i: (i, 0)),
            pl.BlockSpec((blk, 128), lambda i: (i, 0)),
            pl.BlockSpec((128, H), lambda i: (0, 0)),
            pl.BlockSpec((128, H), lambda i: (0, 0)),
            pl.BlockSpec((1, H), lambda i: (0, 0)),
        ],
        out_specs=pl.BlockSpec((blk, H), lambda i: (i, 0)),
        out_shape=jax.ShapeDtypeStruct((N, H), f32),
    )(s0a, s0b, s1a, s1b, c0, c1, wd_hi, wd_lo, bd)


@_sc_kernel(
    "cnt",
    out_type=jax.ShapeDtypeStruct((NC, N_PAD, 128), f32),
    scratch_types=[
        pltpu.VMEM((128,), jnp.int32),
        pltpu.VMEM((128, 128), f32),
        pltpu.VMEM_SHARED((N_PAD, 128), f32),
    ],
)
def _cnt_kernel(dst_hbm, out_hbm, idx_v, rows_v, acc_sh):
    cid = lax.axis_index("c")
    sid = lax.axis_index("s")

    def zb(i, _):
        for k in range(8):
            rows_v[i, pl.ds(k * 16, 16)] = jnp.zeros((16,), f32)
        return 0
    lax.fori_loop(0, 128, zb, 0)
    base = sid * ROWS_PT
    for off, ln in ((0, 128), (128, 128), (256, 128), (384, 128), (512, 120)):
        pltpu.sync_copy(rows_v.at[pl.ds(0, ln)],
                        acc_sh.at[pl.ds(base + off, ln)])

    plsc.subcore_barrier()

    def ob(i, _):
        for k in range(8):
            rows_v[i, pl.ds(k * 16, 16)] = jnp.ones((16,), f32)
        return 0
    lax.fori_loop(0, 128, ob, 0)

    wid = sid * NC + cid
    n_j = (NCH128 // NW) + jnp.where(wid < NCH128 - (NCH128 // NW) * NW, 1, 0)

    def body(j, _):
        e_base = (wid + j * NW) * 128
        pltpu.sync_copy(dst_hbm.at[pl.ds(e_base, 128)], idx_v)
        pltpu.sync_copy(rows_v, acc_sh.at[idx_v], add=True)
        return 0
    lax.fori_loop(0, n_j, body, 0)

    plsc.subcore_barrier()

    pltpu.sync_copy(acc_sh.at[pl.ds(base, ROWS_PT)],
                    out_hbm.at[cid].at[pl.ds(base, ROWS_PT)])


@_sc_kernel(
    "gather",
    out_type=[
        jax.ShapeDtypeStruct((E_S, 128), jnp.int32),
        jax.ShapeDtypeStruct((E_S, 128), jnp.int32),
    ],
    scratch_types=[
        pltpu.VMEM((128,), jnp.int32),
        pltpu.VMEM((128,), jnp.int32),
        pltpu.VMEM((128,), jnp.int32),
        pltpu.VMEM((128,), jnp.int32),
        pltpu.VMEM((GCH, 128), jnp.int32),
        pltpu.VMEM((GCH, 128), jnp.int32),
        pltpu.VMEM((GCH, 128), jnp.int32),
        pltpu.VMEM((GCH, 128), jnp.int32),
        pltpu.SemaphoreType.DMA,
        pltpu.SemaphoreType.DMA,
        pltpu.SemaphoreType.DMA,
        pltpu.SemaphoreType.DMA,
    ],
)
def _gather_kernel(a_hbm, b_hbm, dst_hbm, src_hbm, ga_hbm, gb_hbm,
                   ixd0, ixd1, ixs0, ixs1, ad0, ad1, bs0, bs1,
                   sg0, sg1, sw0, sw1):
    cid = lax.axis_index("c")
    sid = lax.axis_index("s")
    wid = sid * NC + cid
    ngc = dst_hbm.shape[0] // GCH
    n_t = (ngc // NW) + jnp.where(wid < ngc - (ngc // NW) * NW, 1, 0)
    kmax = -(-ngc // NW)

    ixd = (ixd0, ixd1)
    ixs = (ixs0, ixs1)
    ad = (ad0, ad1)
    bs = (bs0, bs1)
    sg = (sg0, sg1)
    sw = (sw0, sw1)

    def drain(sem, n):
        for _ in range(n):
            pltpu.make_async_copy(ga_hbm.at[pl.ds(0, GCH)], ad0, sem).wait()

    def slot(k, p, fire_new):
        @pl.when((k >= 2) & (k - 2 < n_t))
        def _():
            drain(sw[p], 2)

        if fire_new:
            @pl.when(k < n_t)
            def _():
                base_e = (wid + k * NW) * GCH
                pltpu.sync_copy(dst_hbm.at[pl.ds(base_e, GCH)], ixd[p])
                pltpu.sync_copy(src_hbm.at[pl.ds(base_e, GCH)], ixs[p])
                pltpu.async_copy(a_hbm.at[ixd[p]], ad[p], sg[p])
                pltpu.async_copy(b_hbm.at[ixs[p]], bs[p], sg[p])

        q = 1 - p

        @pl.when((k >= 1) & (k - 1 < n_t))
        def _():
            drain(sg[q], 2)
            base_e = (wid + (k - 1) * NW) * GCH
            pltpu.async_copy(ad[q], ga_hbm.at[pl.ds(base_e, GCH)], sw[q])
            pltpu.async_copy(bs[q], gb_hbm.at[pl.ds(base_e, GCH)], sw[q])

    def body(u, _):
        slot(2 * u, 0, True)
        slot(2 * u + 1, 1, True)
        return 0

    npair = (kmax + 1) // 2
    lax.fori_loop(0, npair, body, 0)
    slot(2 * npair, 0, False)
    slot(2 * npair + 1, 1, False)


@_sc_kernel(
    "scatter",
    out_type=[
        jax.ShapeDtypeStruct((N, 128), f32),
        jax.ShapeDtypeStruct((N, 128), f32),
    ],
    scratch_types=[
        pltpu.VMEM((128,), jnp.int32),
        pltpu.VMEM((128,), jnp.int32),
        pltpu.VMEM((128, 128), f32),
        pltpu.VMEM((128, 128), f32),
        pltpu.VMEM_SHARED((N_PAD, 128), f32),
        pltpu.SemaphoreType.DMA,
        pltpu.SemaphoreType.DMA,
        pltpu.SemaphoreType.DMA,
        pltpu.SemaphoreType.DMA,
    ],
)
def _scatter_kernel(m2a_hbm, m2b_hbm, dst_hbm, s0_hbm, s1_hbm,
                    ix0, ix1, rv0, rv1, acc_sh, sl0, sl1, ss0, ss1):
    cid = lax.axis_index("c")
    sid = lax.axis_index("s")

    ix = (ix0, ix1)
    rv = (rv0, rv1)
    sl = (sl0, sl1)
    ss = (ss0, ss1)

    def zb(i, _):
        for k in range(8):
            rv0[i, pl.ds(k * 16, 16)] = jnp.zeros((16,), f32)
        return 0
    lax.fori_loop(0, 128, zb, 0)
    base = sid * ROWS_PT
    for off, ln in ((0, 128), (128, 128), (256, 128), (384, 128), (512, 120)):
        pltpu.sync_copy(rv0.at[pl.ds(0, ln)],
                        acc_sh.at[pl.ds(base + off, ln)])

    plsc.subcore_barrier()

    nch = m2a_hbm.shape[0] // 128
    n_j = (nch // NS) + jnp.where(sid < nch - (nch // NS) * NS, 1, 0)
    kmax = -(-nch // NS)

    def work(m2_ref):
        def drain_rows(sem):
            pltpu.make_async_copy(m2_ref.at[pl.ds(0, 128)], rv0, sem).wait()

        def drain_idx(sem):
            pltpu.make_async_copy(dst_hbm.at[pl.ds(0, 128)], ix0, sem).wait()

        def slot(k, p, fire_new):
            @pl.when((k >= 2) & (k - 2 < n_j))
            def _():
                drain_rows(ss[p])

            if fire_new:
                @pl.when(k < n_j)
                def _():
                    e_base = (sid + k * NS) * 128
                    pltpu.async_copy(dst_hbm.at[pl.ds(e_base, 128)], ix[p], sl[p])
                    pltpu.async_copy(m2_ref.at[pl.ds(e_base, 128)], rv[p], sl[p])

            q = 1 - p

            @pl.when((k >= 1) & (k - 1 < n_j))
            def _():
                drain_idx(sl[q])
                drain_rows(sl[q])
                pltpu.async_copy(rv[q], acc_sh.at[ix[q]], ss[q], add=True)

        def body(u, _):
            slot(2 * u, 0, True)
            slot(2 * u + 1, 1, True)
            return 0

        npair = (kmax + 1) // 2
        lax.fori_loop(0, npair, body, 0)
        slot(2 * npair, 0, False)
        slot(2 * npair + 1, 1, False)

    @pl.when(cid == 0)
    def _():
        work(m2a_hbm)

    @pl.when(cid == 1)
    def _():
        work(m2b_hbm)

    plsc.subcore_barrier()

    def writeback(s_hbm):
        @pl.when(sid < NS - 1)
        def _():
            pltpu.sync_copy(acc_sh.at[pl.ds(base, ROWS_PT)],
                            s_hbm.at[pl.ds(base, ROWS_PT)])

        @pl.when(sid == NS - 1)
        def _():
            ln = N - (NS - 1) * ROWS_PT
            pltpu.sync_copy(acc_sh.at[pl.ds((NS - 1) * ROWS_PT, ln)],
                            s_hbm.at[pl.ds((NS - 1) * ROWS_PT, ln)])

    @pl.when(cid == 0)
    def _():
        writeback(s0_hbm)

    @pl.when(cid == 1)
    def _():
        writeback(s1_hbm)


def kernel(x, edge_index, W_enc, b_enc,
           l0_W1, l0_b1, l0_W2, l0_b2,
           l1_W1, l1_b1, l1_W2, l1_b2,
           l2_W1, l2_b1, l2_W2, l2_b2,
           W_dec, b_dec):
    src1 = edge_index[0]
    dst1 = edge_index[1]

    layers = [(l0_W1, l0_b1, l0_W2, l0_b2),
              (l1_W1, l1_b1, l1_W2, l1_b2),
              (l2_W1, l2_b1, l2_W2, l2_b2)]

    craw = _cnt_kernel(dst1)
    c0 = craw[0, :N]
    c1 = craw[1, :N]

    A, B = _enc_ab(x, W_enc.astype(bf16), b_enc.reshape(1, H),
                   l0_W1[:H].astype(bf16), l0_W1[H:].astype(bf16),
                   l0_b1.reshape(1, H))

    dsts = [lax.slice_in_dim(dst1, s * E_S, (s + 1) * E_S) for s in range(NSLAB)]
    srcs = [lax.slice_in_dim(src1, s * E_S, (s + 1) * E_S) for s in range(NSLAB)]

    out = None
    for l, (W1, b1, W2, b2) in enumerate(layers):
        w2b = W2.astype(bf16)
        b2r = b2.reshape(1, H)
        parts = []
        for s in range(NSLAB):
            ga_i, gb_i = _gather_kernel(A, B, dsts[s], srcs[s])
            m2a, m2b = _mlp(ga_i, gb_i, w2b, b2r)
            parts.append(_scatter_kernel(m2a, m2b, dsts[s]))
        (s0a, s1a), (s0b, s1b) = parts
        if l == 2:
            out = _dec(s0a, s0b, s1a, s1b, c0, c1, W_dec[:128], W_dec[128:],
                       b_dec.reshape(1, H))
        else:
            nW1, nb1 = layers[l + 1][0], layers[l + 1][1]
            A, B = _agg_ab(s0a, s0b, s1a, s1b, c0, c1,
                           nW1[:128].astype(bf16), nW1[128:H].astype(bf16),
                           nW1[H:H + 128].astype(bf16), nW1[H + 128:].astype(bf16),
                           nb1.reshape(1, H))
    return out
